# Initial kernel scaffold; baseline (speedup 1.0000x reference)
import functools

import jax
import jax.numpy as jnp
from jax import lax
from jax.experimental import pallas as pl
from jax.experimental.pallas import tpu as pltpu
from jax.experimental.pallas import tpu_sc as plsc

N_NODES = 10000
N_EDGES = 160000
D_FEAT = 256
HALF = 128
NC = 2
NS = 16
EDGES_PER_TILE = N_EDGES // NS
CHUNK = 80
NCHUNK = EDGES_PER_TILE // CHUNK
ROWS_PER_TILE = N_NODES // NS
ZROWS = 125


def _body(x_hbm, src_hbm, dst_hbm, out_hbm,
          src_all, dst_all, idx_buf, dst_buf, rows, zbuf, acc, sem):
    c = lax.axis_index("c")
    s = lax.axis_index("s")

    ebase = pl.multiple_of(s * EDGES_PER_TILE, 8)
    pltpu.sync_copy(src_hbm.at[pl.ds(ebase, EDGES_PER_TILE)], src_all)
    pltpu.sync_copy(dst_hbm.at[pl.ds(ebase, EDGES_PER_TILE)], dst_all)

    zero16 = jnp.zeros((16,), jnp.float32)

    def zrow(r, carry):
        for k in range(HALF // 16):
            zbuf[r, pl.ds(k * 16, 16)] = zero16
        return carry
    lax.fori_loop(0, ZROWS, zrow, 0)
    obase = s * ROWS_PER_TILE
    for k in range(ROWS_PER_TILE // ZROWS):
        pltpu.sync_copy(zbuf, acc.at[pl.ds(obase + k * ZROWS, ZROWS)])
    plsc.subcore_barrier()

    def chunk(j, carry):
        off = j * CHUNK
        for k in range(CHUNK // 16):
            sv = src_all[pl.ds(off + k * 16, 16)]
            idx_buf[0, pl.ds(k * 16, 16)] = sv * 2 + c
            dst_buf[0, pl.ds(k * 16, 16)] = dst_all[pl.ds(off + k * 16, 16)]
        pltpu.async_copy(x_hbm.at[idx_buf.at[0]], rows.at[0], sem).wait()
        pltpu.sync_copy(rows.at[0], acc.at[dst_buf.at[0]], add=True)
        return carry
    lax.fori_loop(0, NCHUNK, chunk, 0)

    plsc.subcore_barrier()
    cbase = pl.multiple_of(c * HALF, HALF)
    pltpu.sync_copy(acc.at[pl.ds(obase, ROWS_PER_TILE)],
                    out_hbm.at[pl.ds(obase, ROWS_PER_TILE), pl.ds(cbase, HALF)])


_mesh = plsc.VectorSubcoreMesh(core_axis_name="c", subcore_axis_name="s")

_sc_call = pl.kernel(
    _body,
    out_type=jax.ShapeDtypeStruct((N_NODES, D_FEAT), jnp.float32),
    mesh=_mesh,
    scratch_types=[
        pltpu.VMEM((EDGES_PER_TILE,), jnp.int32),
        pltpu.VMEM((EDGES_PER_TILE,), jnp.int32),
        pltpu.VMEM((2, CHUNK), jnp.int32),
        pltpu.VMEM((2, CHUNK), jnp.int32),
        pltpu.VMEM((2, CHUNK, HALF), jnp.float32),
        pltpu.VMEM((ZROWS, HALF), jnp.float32),
        pltpu.VMEM_SHARED((N_NODES, HALF), jnp.float32),
        pltpu.SemaphoreType.DMA,
    ],
)


def kernel(x, edge_index):
    x_r = x.reshape(2 * N_NODES, HALF)
    src = edge_index[0]
    dst = edge_index[1]
    return _sc_call(x_r, src, dst)


# speedup vs baseline: 5.3573x; 5.3573x over previous
import functools

import jax
import jax.numpy as jnp
from jax import lax
from jax.experimental import pallas as pl
from jax.experimental.pallas import tpu as pltpu
from jax.experimental.pallas import tpu_sc as plsc

N_NODES = 10000
N_EDGES = 160000
D_FEAT = 256
HALF = 128
NC = 2
NS = 16
EDGES_PER_TILE = N_EDGES // NS
CHUNK = 80
NCHUNK = EDGES_PER_TILE // CHUNK
ROWS_PER_TILE = N_NODES // NS
ZROWS = 25
WSLAB = 624


def _body(x_hbm, src_hbm, dst_hbm, out_hbm,
          src_all, dst_all, idx_buf, dst_buf, rows, zbuf, acc, sem):
    c = lax.axis_index("c")
    s = lax.axis_index("s")

    ebase = pl.multiple_of(s * EDGES_PER_TILE, 8)
    pltpu.sync_copy(src_hbm.at[pl.ds(ebase, EDGES_PER_TILE)], src_all)
    pltpu.sync_copy(dst_hbm.at[pl.ds(ebase, EDGES_PER_TILE)], dst_all)

    zero16 = jnp.zeros((16,), jnp.float32)

    def zrow(r, carry):
        for k in range(HALF // 16):
            zbuf[r, pl.ds(k * 16, 16)] = zero16
        return carry
    lax.fori_loop(0, ZROWS, zrow, 0)
    obase = s * ROWS_PER_TILE
    for k in range(ROWS_PER_TILE // ZROWS):
        pltpu.sync_copy(zbuf, acc.at[pl.ds(obase + k * ZROWS, ZROWS)])
    plsc.subcore_barrier()

    def chunk(j, carry):
        off = j * CHUNK
        for k in range(CHUNK // 16):
            sv = src_all[pl.ds(off + k * 16, 16)]
            idx_buf[0, pl.ds(k * 16, 16)] = sv * 2 + c
            dst_buf[0, pl.ds(k * 16, 16)] = dst_all[pl.ds(off + k * 16, 16)]
        pltpu.async_copy(x_hbm.at[idx_buf.at[0]], rows.at[0], sem).wait()
        pltpu.sync_copy(rows.at[0], acc.at[dst_buf.at[0]], add=True)
        return carry
    lax.fori_loop(0, NCHUNK, chunk, 0)

    plsc.subcore_barrier()
    cbase = pl.multiple_of(c * HALF, HALF)
    wbase = pl.multiple_of(s * WSLAB, 8)
    pltpu.sync_copy(acc.at[pl.ds(wbase, WSLAB)],
                    out_hbm.at[pl.ds(wbase, WSLAB), pl.ds(cbase, HALF)])

    @pl.when(s == NS - 1)
    def _tail():
        tbase = NS * WSLAB
        pltpu.sync_copy(acc.at[pl.ds(tbase, N_NODES - NS * WSLAB)],
                        out_hbm.at[pl.ds(tbase, N_NODES - NS * WSLAB),
                                   pl.ds(cbase, HALF)])


_mesh = plsc.VectorSubcoreMesh(core_axis_name="c", subcore_axis_name="s")

_sc_call = pl.kernel(
    _body,
    out_type=jax.ShapeDtypeStruct((N_NODES, D_FEAT), jnp.float32),
    mesh=_mesh,
    scratch_types=[
        pltpu.VMEM((EDGES_PER_TILE,), jnp.int32),
        pltpu.VMEM((EDGES_PER_TILE,), jnp.int32),
        pltpu.VMEM((2, CHUNK), jnp.int32),
        pltpu.VMEM((2, CHUNK), jnp.int32),
        pltpu.VMEM((2, CHUNK, HALF), jnp.float32),
        pltpu.VMEM((ZROWS, HALF), jnp.float32),
        pltpu.VMEM_SHARED((N_NODES, HALF), jnp.float32),
        pltpu.SemaphoreType.DMA,
    ],
)


def kernel(x, edge_index):
    x_r = x.reshape(2 * N_NODES, HALF)
    src = edge_index[0]
    dst = edge_index[1]
    return _sc_call(x_r, src, dst)


# speedup vs baseline: 8.5410x; 1.5943x over previous
import functools

import jax
import jax.numpy as jnp
from jax import lax
from jax.experimental import pallas as pl
from jax.experimental.pallas import tpu as pltpu
from jax.experimental.pallas import tpu_sc as plsc

N_NODES = 10000
N_EDGES = 160000
D_FEAT = 256
HALF = 128
NC = 2
NS = 16
EDGES_PER_TILE = N_EDGES // NS
CHUNK = 80
NCHUNK = EDGES_PER_TILE // CHUNK
ROWS_PER_TILE = N_NODES // NS
ZROWS = 25
WSLAB = 624


def _body(x_hbm, src_hbm, dst_hbm, out_hbm,
          src_all, dst_all, idx_buf, dst_buf, rows, zbuf, acc, sem0, sem1):
    c = lax.axis_index("c")
    s = lax.axis_index("s")

    ebase = pl.multiple_of(s * EDGES_PER_TILE, 8)
    pltpu.sync_copy(src_hbm.at[pl.ds(ebase, EDGES_PER_TILE)], src_all)
    pltpu.sync_copy(dst_hbm.at[pl.ds(ebase, EDGES_PER_TILE)], dst_all)

    zero16 = jnp.zeros((16,), jnp.float32)

    def zrow(r, carry):
        for k in range(HALF // 16):
            zbuf[r, pl.ds(k * 16, 16)] = zero16
        return carry
    lax.fori_loop(0, ZROWS, zrow, 0)
    obase = s * ROWS_PER_TILE
    for k in range(ROWS_PER_TILE // ZROWS):
        pltpu.sync_copy(zbuf, acc.at[pl.ds(obase + k * ZROWS, ZROWS)])
    plsc.subcore_barrier()

    sems = [sem0, sem1]

    def fill_idx(j, b):
        off = j * CHUNK
        for k in range(CHUNK // 16):
            sv = src_all[pl.ds(off + k * 16, 16)]
            idx_buf[b, pl.ds(k * 16, 16)] = sv * 2 + c
            dst_buf[b, pl.ds(k * 16, 16)] = dst_all[pl.ds(off + k * 16, 16)]

    def fire(b):
        pltpu.async_copy(x_hbm.at[idx_buf.at[b]], rows.at[b], sems[b])

    def drain_scatter(b):
        pltpu.make_async_copy(x_hbm.at[idx_buf.at[b]], rows.at[b],
                              sems[b]).wait()
        pltpu.sync_copy(rows.at[b], acc.at[dst_buf.at[b]], add=True)

    fill_idx(0, 0)
    fire(0)

    def dbl(i, carry):
        fill_idx(2 * i + 1, 1)
        fire(1)
        drain_scatter(0)
        fill_idx(2 * i + 2, 0)
        fire(0)
        drain_scatter(1)
        return carry
    lax.fori_loop(0, (NCHUNK - 1) // 2, dbl, 0)
    drain_scatter(0)

    plsc.subcore_barrier()
    cbase = pl.multiple_of(c * HALF, HALF)
    wbase = pl.multiple_of(s * WSLAB, 8)
    pltpu.sync_copy(acc.at[pl.ds(wbase, WSLAB)],
                    out_hbm.at[pl.ds(wbase, WSLAB), pl.ds(cbase, HALF)])

    @pl.when(s == NS - 1)
    def _tail():
        tbase = NS * WSLAB
        pltpu.sync_copy(acc.at[pl.ds(tbase, N_NODES - NS * WSLAB)],
                        out_hbm.at[pl.ds(tbase, N_NODES - NS * WSLAB),
                                   pl.ds(cbase, HALF)])


_mesh = plsc.VectorSubcoreMesh(core_axis_name="c", subcore_axis_name="s")

_sc_call = pl.kernel(
    _body,
    out_type=jax.ShapeDtypeStruct((N_NODES, D_FEAT), jnp.float32),
    mesh=_mesh,
    scratch_types=[
        pltpu.VMEM((EDGES_PER_TILE,), jnp.int32),
        pltpu.VMEM((EDGES_PER_TILE,), jnp.int32),
        pltpu.VMEM((2, CHUNK), jnp.int32),
        pltpu.VMEM((2, CHUNK), jnp.int32),
        pltpu.VMEM((2, CHUNK, HALF), jnp.float32),
        pltpu.VMEM((ZROWS, HALF), jnp.float32),
        pltpu.VMEM_SHARED((N_NODES, HALF), jnp.float32),
        pltpu.SemaphoreType.DMA,
        pltpu.SemaphoreType.DMA,
    ],
)


def kernel(x, edge_index):
    x_r = x.reshape(2 * N_NODES, HALF)
    src = edge_index[0]
    dst = edge_index[1]
    return _sc_call(x_r, src, dst)
